# Initial kernel scaffold; baseline (speedup 1.0000x reference)
import jax
import jax.numpy as jnp
from jax.experimental import pallas as pl


def kernel(x, idx0, idx1, W0, b0, W1, b1, W2, b2):
    raise NotImplementedError("write your pallas kernel here")


# speedup vs baseline: 2.6084x; 2.6084x over previous
import functools

import jax
import jax.numpy as jnp
from jax import lax
from jax.experimental import pallas as pl
from jax.experimental.pallas import tpu as pltpu
from jax.experimental.pallas import tpu_sc as plsc

_SEQ = 9
_NPAD = 53248
_BLK = 128


def _mm(x, w, b, bn=512):
    n, k = x.shape
    m = w.shape[1]

    def body(x_ref, w_ref, b_ref, o_ref):
        o_ref[...] = jnp.dot(x_ref[...], w_ref[...],
                             preferred_element_type=jnp.float32) + b_ref[...]

    return pl.pallas_call(
        body,
        grid=(n // bn,),
        in_specs=[
            pl.BlockSpec((bn, k), lambda i: (i, 0)),
            pl.BlockSpec((k, m), lambda i: (0, 0)),
            pl.BlockSpec((1, m), lambda i: (0, 0)),
        ],
        out_specs=pl.BlockSpec((bn, m), lambda i: (i, 0)),
        out_shape=jax.ShapeDtypeStruct((n, m), jnp.float32),
    )(x, w, b)


def _make_gather_sum(D, table_rows):
    info = plsc.get_sparse_core_info()
    nc, ns = info.num_cores, info.num_subcores
    nw = nc * ns
    nodes_per_w = _NPAD // nw
    nblk = nodes_per_w // _BLK
    idx_rows_per_blk = _BLK * _SEQ // 128
    mesh = plsc.VectorSubcoreMesh(core_axis_name="c", subcore_axis_name="s")

    @functools.partial(
        pl.kernel, mesh=mesh,
        compiler_params=pltpu.CompilerParams(use_tc_tiling_on_sc=False),
        out_type=jax.ShapeDtypeStruct((_NPAD, D), jnp.float32),
        scratch_types=[
            pltpu.VMEM((_BLK * _SEQ,), jnp.int32),
            pltpu.VMEM((idx_rows_per_blk, 128), jnp.int32),
            pltpu.VMEM((_BLK * _SEQ, D), jnp.float32),
            pltpu.VMEM((_BLK, D), jnp.float32),
            pltpu.SemaphoreType.DMA,
        ])
    def gsum(idx_hbm, table_hbm, out_hbm, idxv, fiv, rows, ob, sem):
        wid = lax.axis_index("s") * nc + lax.axis_index("c")

        def block_body(b, carry):
            node0 = wid * nodes_per_w + b * _BLK
            pltpu.sync_copy(idx_hbm.at[pl.ds(node0 * _SEQ, _BLK * _SEQ)], idxv)
            for j in range(_BLK * _SEQ // 16):
                r, c = j // 8, (j % 8) * 16
                sl = pl.ds(c, 16)
                pos = lax.iota(jnp.int32, 16) + (16 * j)
                fiv[r, sl] = idxv[pl.ds(16 * j, 16)] * _SEQ + lax.rem(pos, _SEQ)
            cps = []
            for ch in range(idx_rows_per_blk):
                cps.append(pltpu.async_copy(
                    table_hbm.at[fiv.at[ch]],
                    rows.at[pl.ds(ch * 128, 128)], sem))
            for cp in cps:
                cp.wait()

            def node_body(n, _):
                r0 = n * _SEQ
                for c0 in range(0, D, 16):
                    acc = rows[r0, pl.ds(c0, 16)]
                    for k in range(1, _SEQ):
                        acc = acc + rows[r0 + k, pl.ds(c0, 16)]
                    ob[n, pl.ds(c0, 16)] = acc
                return 0

            lax.fori_loop(0, _BLK, node_body, 0)
            pltpu.sync_copy(ob, out_hbm.at[pl.ds(node0, _BLK)])
            return carry

        lax.fori_loop(0, nblk, block_body, 0)

    return gsum


def kernel(x, idx0, idx1, W0, b0, W1, b1, W2, b2):
    n, in_c = x.shape
    c0 = W0.shape[1]
    c1 = W1.shape[1]
    pad = _NPAD - n

    xp = jnp.pad(x, ((0, pad), (0, 0)))
    idx0f = jnp.pad(idx0, ((0, pad), (0, 0))).reshape(-1)
    idx1f = jnp.pad(idx1, ((0, pad), (0, 0))).reshape(-1)

    w0r = W0.reshape(_SEQ, in_c, c0).transpose(1, 0, 2).reshape(in_c, _SEQ * c0)
    bias0 = (jnp.tile(b0, _SEQ) / _SEQ).reshape(1, -1)
    p = _mm(xp, w0r, bias0)
    h = _make_gather_sum(c0, _NPAD * _SEQ)(idx0f, p.reshape(-1, c0))

    w1r = W1.reshape(_SEQ, c0, c1).transpose(1, 0, 2).reshape(c0, _SEQ * c1)
    bias1 = (jnp.tile(b1, _SEQ) / _SEQ).reshape(1, -1)
    q = _mm(h, w1r, bias1)
    h2 = _make_gather_sum(c1, _NPAD * _SEQ)(idx1f, q.reshape(-1, c1))

    out = _mm(h2, W2, b2.reshape(1, -1))
    return out[:n]


# speedup vs baseline: 3.0780x; 1.1800x over previous
import functools

import jax
import jax.numpy as jnp
from jax import lax
from jax.experimental import pallas as pl
from jax.experimental.pallas import tpu as pltpu
from jax.experimental.pallas import tpu_sc as plsc

_SEQ = 9
_NPAD = 53248
_BLK = 128


def _mm(x, w, b, bn=512):
    n, k = x.shape
    m = w.shape[1]

    def body(x_ref, w_ref, b_ref, o_ref):
        o_ref[...] = jnp.dot(x_ref[...], w_ref[...],
                             preferred_element_type=jnp.float32) + b_ref[...]

    return pl.pallas_call(
        body,
        grid=(n // bn,),
        in_specs=[
            pl.BlockSpec((bn, k), lambda i: (i, 0)),
            pl.BlockSpec((k, m), lambda i: (0, 0)),
            pl.BlockSpec((1, m), lambda i: (0, 0)),
        ],
        out_specs=pl.BlockSpec((bn, m), lambda i: (i, 0)),
        out_shape=jax.ShapeDtypeStruct((n, m), jnp.float32),
    )(x, w, b)


def _make_gather_sum(D, table_rows):
    info = plsc.get_sparse_core_info()
    nc, ns = info.num_cores, info.num_subcores
    nw = nc * ns
    nodes_per_w = _NPAD // nw
    nblk = nodes_per_w // _BLK
    idx_rows_per_blk = _BLK * _SEQ // 128
    mesh = plsc.VectorSubcoreMesh(core_axis_name="c", subcore_axis_name="s")

    @functools.partial(
        pl.kernel, mesh=mesh,
        compiler_params=pltpu.CompilerParams(use_tc_tiling_on_sc=False),
        out_type=jax.ShapeDtypeStruct((_NPAD, D), jnp.float32),
        scratch_types=[
            pltpu.VMEM((_BLK * _SEQ,), jnp.int32),
            pltpu.VMEM((idx_rows_per_blk, 128), jnp.int32),
            pltpu.VMEM((_BLK * _SEQ, D), jnp.float32),
            pltpu.VMEM((_BLK, D), jnp.float32),
            pltpu.SemaphoreType.DMA,
        ])
    def gsum(idx_hbm, table_hbm, out_hbm, idxv, fiv, rows, ob, sem):
        wid = lax.axis_index("s") * nc + lax.axis_index("c")

        def block_body(b, carry):
            node0 = wid * nodes_per_w + b * _BLK
            pltpu.sync_copy(idx_hbm.at[pl.ds(node0 * _SEQ, _BLK * _SEQ)], idxv)
            for j in range(_BLK * _SEQ // 16):
                r, c = j // 8, (j % 8) * 16
                sl = pl.ds(c, 16)
                pos = lax.iota(jnp.int32, 16) + (16 * j)
                fiv[r, sl] = idxv[pl.ds(16 * j, 16)] * _SEQ + lax.rem(pos, _SEQ)
            cps = []
            for ch in range(idx_rows_per_blk):
                cps.append(pltpu.async_copy(
                    table_hbm.at[fiv.at[ch]],
                    rows.at[pl.ds(ch * 128, 128)], sem))
            for cp in cps:
                cp.wait()

            def node_body(n, _):
                r0 = n * _SEQ
                for c0 in range(0, D, 16):
                    acc = rows[r0, pl.ds(c0, 16)]
                    for k in range(1, _SEQ):
                        acc = acc + rows[r0 + k, pl.ds(c0, 16)]
                    ob[n, pl.ds(c0, 16)] = acc
                return 0

            lax.fori_loop(0, _BLK, node_body, 0)
            pltpu.sync_copy(ob, out_hbm.at[pl.ds(node0, _BLK)])
            return carry

        lax.fori_loop(0, nblk, block_body, 0)

    return gsum


def kernel(x, idx0, idx1, W0, b0, W1, b1, W2, b2):
    n, in_c = x.shape
    c0 = W0.shape[1]
    c1 = W1.shape[1]
    pad = _NPAD - n

    lat = W2.shape[1]

    xp = jnp.pad(x, ((0, pad), (0, 0)))
    idx0f = jnp.pad(idx0, ((0, pad), (0, 0))).reshape(-1)
    idx1f = jnp.pad(idx1, ((0, pad), (0, 0))).reshape(-1)

    w0r = W0.reshape(_SEQ, in_c, c0).transpose(1, 0, 2).reshape(in_c, _SEQ * c0)
    bias0 = (jnp.tile(b0, _SEQ) / _SEQ).reshape(1, -1)
    p = _mm(xp, w0r, bias0, bn=1024)
    h = _make_gather_sum(c0, _NPAD * _SEQ)(idx0f, p.reshape(-1, c0))

    w12 = jnp.einsum("sck,kl->scl", W1.reshape(_SEQ, c0, c1), W2)
    w12 = jnp.pad(w12, ((0, 0), (0, 0), (0, 16 - lat)))
    w12r = w12.transpose(1, 0, 2).reshape(c0, _SEQ * 16)
    b12 = jnp.pad(b1 @ W2 + b2, (0, 16 - lat))
    bias12 = (jnp.tile(b12, _SEQ) / _SEQ).reshape(1, -1)
    q = _mm(h, w12r, bias12, bn=2048)
    h2 = _make_gather_sum(16, _NPAD * _SEQ)(idx1f, q.reshape(-1, 16))

    return h2[:n, :lat]


# speedup vs baseline: 3.2288x; 1.0490x over previous
import functools

import jax
import jax.numpy as jnp
from jax import lax
from jax.experimental import pallas as pl
from jax.experimental.pallas import tpu as pltpu
from jax.experimental.pallas import tpu_sc as plsc

_SEQ = 9
_NPAD = 53248
_BLK = 128


def _mm(x, w, b, bn=512):
    n, k = x.shape
    m = w.shape[1]

    def body(x_ref, w_ref, b_ref, o_ref):
        o_ref[...] = jnp.dot(x_ref[...], w_ref[...],
                             preferred_element_type=jnp.float32) + b_ref[...]

    return pl.pallas_call(
        body,
        grid=(n // bn,),
        in_specs=[
            pl.BlockSpec((bn, k), lambda i: (i, 0)),
            pl.BlockSpec((k, m), lambda i: (0, 0)),
            pl.BlockSpec((1, m), lambda i: (0, 0)),
        ],
        out_specs=pl.BlockSpec((bn, m), lambda i: (i, 0)),
        out_shape=jax.ShapeDtypeStruct((n, m), jnp.float32),
    )(x, w, b)


def _make_gather_sum(D, table_rows):
    info = plsc.get_sparse_core_info()
    nc, ns = info.num_cores, info.num_subcores
    nw = nc * ns
    nodes_per_w = _NPAD // nw
    nblk = nodes_per_w // _BLK
    idx_rows_per_blk = _BLK * _SEQ // 128
    mesh = plsc.VectorSubcoreMesh(core_axis_name="c", subcore_axis_name="s")

    @functools.partial(
        pl.kernel, mesh=mesh,
        compiler_params=pltpu.CompilerParams(use_tc_tiling_on_sc=False),
        out_type=jax.ShapeDtypeStruct((_NPAD, D), jnp.float32),
        scratch_types=[
            pltpu.VMEM((_BLK * _SEQ,), jnp.int32),
            pltpu.VMEM((_BLK * _SEQ,), jnp.int32),
            pltpu.VMEM((idx_rows_per_blk, 128), jnp.int32),
            pltpu.VMEM((idx_rows_per_blk, 128), jnp.int32),
            pltpu.VMEM((_BLK * _SEQ, D), jnp.float32),
            pltpu.VMEM((_BLK * _SEQ, D), jnp.float32),
            pltpu.VMEM((_BLK, D), jnp.float32),
            pltpu.VMEM((_BLK, D), jnp.float32),
            pltpu.SemaphoreType.DMA,
            pltpu.SemaphoreType.DMA,
            pltpu.SemaphoreType.DMA,
            pltpu.SemaphoreType.DMA,
        ])
    def gsum(idx_hbm, table_hbm, out_hbm,
             idxv0, idxv1, fiv0, fiv1, rows0, rows1, ob0, ob1,
             sg0, sg1, so0, so1):
        wid = lax.axis_index("s") * nc + lax.axis_index("c")
        base_node = wid * nodes_per_w

        def fire(b, idxv, fiv, rows, sg):
            node0 = base_node + b * _BLK
            pltpu.sync_copy(idx_hbm.at[pl.ds(node0 * _SEQ, _BLK * _SEQ)], idxv)
            for j in range(_BLK * _SEQ // 16):
                r, c = j // 8, (j % 8) * 16
                pos = lax.iota(jnp.int32, 16) + (16 * j)
                fiv[r, pl.ds(c, 16)] = (idxv[pl.ds(16 * j, 16)] * _SEQ
                                        + lax.rem(pos, _SEQ))
            for ch in range(idx_rows_per_blk):
                pltpu.async_copy(table_hbm.at[fiv.at[ch]],
                                 rows.at[pl.ds(ch * 128, 128)], sg)

        def drain_rows(rows, sg):
            pltpu.make_async_copy(
                table_hbm.at[pl.ds(0, _BLK * _SEQ)], rows, sg).wait()

        def drain_ob(ob, so):
            pltpu.make_async_copy(table_hbm.at[pl.ds(0, _BLK)], ob, so).wait()

        def consume(b, rows, ob, so):
            node0 = base_node + b * _BLK

            def node_body(m, _):
                for u in range(4):
                    n = m * 4 + u
                    r0 = n * _SEQ
                    for c0 in range(0, D, 16):
                        acc = rows[r0, pl.ds(c0, 16)]
                        for k in range(1, _SEQ):
                            acc = acc + rows[r0 + k, pl.ds(c0, 16)]
                        ob[n, pl.ds(c0, 16)] = acc
                return 0

            lax.fori_loop(0, _BLK // 4, node_body, 0)
            pltpu.async_copy(ob, out_hbm.at[pl.ds(node0, _BLK)], so)

        bufs0 = (idxv0, fiv0, rows0, sg0, ob0, so0)
        bufs1 = (idxv1, fiv1, rows1, sg1, ob1, so1)

        fire(0, idxv0, fiv0, rows0, sg0)

        def loop_body(b, carry):
            def stage(cur, nxt):
                idxn, fin, rn, sgn, _, _ = nxt
                _, _, rc, sgc, obc, soc = cur

                @pl.when(b + 1 < nblk)
                def _():
                    fire(b + 1, idxn, fin, rn, sgn)

                drain_rows(rc, sgc)

                @pl.when(b >= 2)
                def _():
                    drain_ob(obc, soc)

                consume(b, rc, obc, soc)

            @pl.when(lax.rem(b, 2) == 0)
            def _():
                stage(bufs0, bufs1)

            @pl.when(lax.rem(b, 2) == 1)
            def _():
                stage(bufs1, bufs0)

            return carry

        lax.fori_loop(0, nblk, loop_body, 0)
        drain_ob(ob0, so0)
        drain_ob(ob1, so1)

    return gsum


def kernel(x, idx0, idx1, W0, b0, W1, b1, W2, b2):
    n, in_c = x.shape
    c0 = W0.shape[1]
    c1 = W1.shape[1]
    pad = _NPAD - n

    lat = W2.shape[1]

    xp = jnp.pad(x, ((0, pad), (0, 0)))
    idx0f = jnp.pad(idx0, ((0, pad), (0, 0))).reshape(-1)
    idx1f = jnp.pad(idx1, ((0, pad), (0, 0))).reshape(-1)

    w0r = W0.reshape(_SEQ, in_c, c0).transpose(1, 0, 2).reshape(in_c, _SEQ * c0)
    bias0 = (jnp.tile(b0, _SEQ) / _SEQ).reshape(1, -1)
    p = _mm(xp, w0r, bias0, bn=1024)
    h = _make_gather_sum(c0, _NPAD * _SEQ)(idx0f, p.reshape(-1, c0))

    w12 = jnp.einsum("sck,kl->scl", W1.reshape(_SEQ, c0, c1), W2)
    w12 = jnp.pad(w12, ((0, 0), (0, 0), (0, 16 - lat)))
    w12r = w12.transpose(1, 0, 2).reshape(c0, _SEQ * 16)
    b12 = jnp.pad(b1 @ W2 + b2, (0, 16 - lat))
    bias12 = (jnp.tile(b12, _SEQ) / _SEQ).reshape(1, -1)
    q = _mm(h, w12r, bias12, bn=2048)
    h2 = _make_gather_sum(16, _NPAD * _SEQ)(idx1f, q.reshape(-1, 16))

    return h2[:n, :lat]


# speedup vs baseline: 5.8038x; 1.7975x over previous
import functools

import jax
import jax.numpy as jnp
from jax import lax
from jax.experimental import pallas as pl
from jax.experimental.pallas import tpu as pltpu
from jax.experimental.pallas import tpu_sc as plsc

_SEQ = 9
_NPAD = 53248
_BLK = 128


def _mm(x, w, b, bn=512):
    n, k = x.shape
    m = w.shape[1]

    def body(x_ref, w_ref, b_ref, o_ref):
        o_ref[...] = jnp.dot(x_ref[...], w_ref[...],
                             preferred_element_type=jnp.float32) + b_ref[...]

    return pl.pallas_call(
        body,
        grid=(n // bn,),
        in_specs=[
            pl.BlockSpec((bn, k), lambda i: (i, 0)),
            pl.BlockSpec((k, m), lambda i: (0, 0)),
            pl.BlockSpec((1, m), lambda i: (0, 0)),
        ],
        out_specs=pl.BlockSpec((bn, m), lambda i: (i, 0)),
        out_shape=jax.ShapeDtypeStruct((n, m), jnp.float32),
    )(x, w, b)


def _make_gather_sum(D, n_nodes):
    info = plsc.get_sparse_core_info()
    nc, ns = info.num_cores, info.num_subcores
    nw = nc * ns
    nblk = -(-n_nodes // (nw * _BLK))
    idx_rows_per_blk = _BLK * _SEQ // 128
    last0 = n_nodes - _BLK
    mesh = plsc.VectorSubcoreMesh(core_axis_name="c", subcore_axis_name="s")

    @functools.partial(
        pl.kernel, mesh=mesh,
        compiler_params=pltpu.CompilerParams(use_tc_tiling_on_sc=False),
        out_type=jax.ShapeDtypeStruct((n_nodes, D), jnp.float32),
        scratch_types=[
            pltpu.VMEM((_BLK * _SEQ,), jnp.int32),
            pltpu.VMEM((_BLK * _SEQ,), jnp.int32),
            pltpu.VMEM((idx_rows_per_blk, 128), jnp.int32),
            pltpu.VMEM((idx_rows_per_blk, 128), jnp.int32),
            pltpu.VMEM((_BLK * _SEQ, D), jnp.float32),
            pltpu.VMEM((_BLK * _SEQ, D), jnp.float32),
            pltpu.VMEM((_BLK, D), jnp.float32),
            pltpu.VMEM((_BLK, D), jnp.float32),
            pltpu.SemaphoreType.DMA,
            pltpu.SemaphoreType.DMA,
            pltpu.SemaphoreType.DMA,
            pltpu.SemaphoreType.DMA,
        ])
    def gsum(idx_hbm, table_hbm, out_hbm,
             idxv0, idxv1, fiv0, fiv1, rows0, rows1, ob0, ob1,
             sg0, sg1, so0, so1):
        wid = lax.axis_index("s") * nc + lax.axis_index("c")

        def node_of(b):
            return jnp.minimum((wid * nblk + b) * _BLK, last0)

        def fire(node0, idxv, fiv, rows, sg):
            pltpu.sync_copy(idx_hbm.at[pl.ds(node0 * _SEQ, _BLK * _SEQ)], idxv)
            for j in range(_BLK * _SEQ // 16):
                r, c = j // 8, (j % 8) * 16
                pos = lax.iota(jnp.int32, 16) + (16 * j)
                fiv[r, pl.ds(c, 16)] = (idxv[pl.ds(16 * j, 16)] * _SEQ
                                        + lax.rem(pos, _SEQ))
            for ch in range(idx_rows_per_blk):
                pltpu.async_copy(table_hbm.at[fiv.at[ch]],
                                 rows.at[pl.ds(ch * 128, 128)], sg)

        def drain_rows(rows, sg):
            pltpu.make_async_copy(
                table_hbm.at[pl.ds(0, _BLK * _SEQ)], rows, sg).wait()

        def drain_ob(ob, so):
            pltpu.make_async_copy(table_hbm.at[pl.ds(0, _BLK)], ob, so).wait()

        def consume(node0, rows, ob, so):
            def node_body(m, _):
                for u in range(4):
                    n = m * 4 + u
                    r0 = n * _SEQ
                    for c0 in range(0, D, 16):
                        acc = rows[r0, pl.ds(c0, 16)]
                        for k in range(1, _SEQ):
                            acc = acc + rows[r0 + k, pl.ds(c0, 16)]
                        ob[n, pl.ds(c0, 16)] = acc
                return 0

            lax.fori_loop(0, _BLK // 4, node_body, 0)
            pltpu.async_copy(ob, out_hbm.at[pl.ds(node0, _BLK)], so)

        bufs0 = (idxv0, fiv0, rows0, sg0, ob0, so0)
        bufs1 = (idxv1, fiv1, rows1, sg1, ob1, so1)

        fire(node_of(0), idxv0, fiv0, rows0, sg0)

        def loop_body(b, carry):
            def stage(cur, nxt):
                idxn, fin, rn, sgn, _, _ = nxt
                _, _, rc, sgc, obc, soc = cur

                @pl.when(b + 1 < nblk)
                def _():
                    fire(node_of(b + 1), idxn, fin, rn, sgn)

                drain_rows(rc, sgc)

                @pl.when(b >= 2)
                def _():
                    drain_ob(obc, soc)

                consume(node_of(b), rc, obc, soc)

            @pl.when(lax.rem(b, 2) == 0)
            def _():
                stage(bufs0, bufs1)

            @pl.when(lax.rem(b, 2) == 1)
            def _():
                stage(bufs1, bufs0)

            return carry

        lax.fori_loop(0, nblk, loop_body, 0)
        drain_ob(ob0, so0)
        drain_ob(ob1, so1)

    return gsum


def kernel(x, idx0, idx1, W0, b0, W1, b1, W2, b2):
    n, in_c = x.shape
    c0 = W0.shape[1]
    c1 = W1.shape[1]
    lat = W2.shape[1]

    w0r = W0.reshape(_SEQ, in_c, c0).transpose(1, 0, 2).reshape(in_c, _SEQ * c0)
    bias0 = (jnp.tile(b0, _SEQ) / _SEQ).reshape(1, -1)
    p = _mm(x, w0r, bias0, bn=2000)
    h = _make_gather_sum(c0, n)(idx0.reshape(-1), p.reshape(-1, c0))

    w12 = jnp.einsum("sck,kl->scl", W1.reshape(_SEQ, c0, c1), W2)
    w12 = jnp.pad(w12, ((0, 0), (0, 0), (0, 16 - lat)))
    w12r = w12.transpose(1, 0, 2).reshape(c0, _SEQ * 16)
    b12 = jnp.pad(b1 @ W2 + b2, (0, 16 - lat))
    bias12 = (jnp.tile(b12, _SEQ) / _SEQ).reshape(1, -1)
    q = _mm(h, w12r, bias12, bn=2000)
    h2 = _make_gather_sum(16, n)(idx1.reshape(-1), q.reshape(-1, 16))
    return h2[:, :lat]
